# baseline (speedup 1.0000x reference)
import functools

import jax
import jax.numpy as jnp
from jax.experimental import pallas as pl
from jax.experimental.pallas import tpu as pltpu

LAT = 128
NNODE = 10000
NEDGE = 320000
EB = 3200


def _ln(h, g, b):
    m = jnp.mean(h, axis=-1, keepdims=True)
    v = jnp.mean((h - m) ** 2, axis=-1, keepdims=True)
    return (h - m) * jax.lax.rsqrt(v + 1e-5) * g + b


def _dot(a, b):
    return jnp.dot(a, b, preferred_element_type=jnp.float32)


def _encode_nodes_body(x, neW1t, neb1, neW2t, neb2, neg, nebe, w1at, w1bt,
                       nl_o, a_o, b_o):
    h = jax.nn.relu(_dot(x[...], neW1t[...]) + neb1[...])
    h = jax.nn.relu(_dot(h, neW2t[...]) + neb2[...])
    nl = _ln(h, neg[...], nebe[...])
    nl_o[...] = nl
    a_o[...] = _dot(nl, w1at[...])
    b_o[...] = _dot(nl, w1bt[...])


def _encode_nodes(x, neW1t, neb1, neW2t, neb2, neg, nebe, w1at, w1bt):
    out = [jax.ShapeDtypeStruct((NNODE, LAT), jnp.float32)] * 3
    return pl.pallas_call(
        _encode_nodes_body,
        out_shape=out,
    )(x, neW1t, neb1, neW2t, neb2, neg, nebe, w1at, w1bt)


def _edge1_body(ea, g1, g2, eeW1t, eeb1, eeW2t, eeb2, eeg, eebe,
                w1ct, b1, w2t, b2, gg, gbe, msg_o, el_o):
    h = jax.nn.relu(_dot(ea[...], eeW1t[...]) + eeb1[...])
    h = jax.nn.relu(_dot(h, eeW2t[...]) + eeb2[...])
    el = _ln(h, eeg[...], eebe[...])
    ce = _dot(el, w1ct[...]) + b1[...]
    m = jax.nn.relu(g1[...] + ce)
    m = jax.nn.relu(_dot(m, w2t[...]) + b2[...])
    msg_o[...] = _ln(m, gg[...], gbe[...])
    n = jax.nn.relu(g2[...] + ce)
    n = jax.nn.relu(_dot(n, w2t[...]) + b2[...])
    el_o[...] = _ln(n, gg[...], gbe[...]) + el


def _edge_step1(ea, g1, g2, eeW1t, eeb1, eeW2t, eeb2, eeg, eebe,
                w1ct, b1, w2t, b2, gg, gbe):
    nblk = NEDGE // EB
    eb_spec = pl.BlockSpec((EB, LAT), lambda i: (i, 0))
    ea_spec = pl.BlockSpec((EB, 16), lambda i: (i, 0))
    w_spec = lambda s: pl.BlockSpec(s, lambda i: (0,) * len(s))
    return pl.pallas_call(
        _edge1_body,
        grid=(nblk,),
        in_specs=[ea_spec, eb_spec, eb_spec,
                  w_spec((16, LAT)), w_spec((1, LAT)), w_spec((LAT, LAT)),
                  w_spec((1, LAT)), w_spec((1, LAT)), w_spec((1, LAT)),
                  w_spec((LAT, LAT)), w_spec((1, LAT)), w_spec((LAT, LAT)),
                  w_spec((1, LAT)), w_spec((1, LAT)), w_spec((1, LAT))],
        out_specs=[eb_spec, eb_spec],
        out_shape=[jax.ShapeDtypeStruct((NEDGE, LAT), jnp.float32)] * 2,
    )(ea, g1, g2, eeW1t, eeb1, eeW2t, eeb2, eeg, eebe,
      w1ct, b1, w2t, b2, gg, gbe)


def _edge2_body(el, g1, w1ct, b1, w2t, b2, gg, gbe, msg_o):
    ce = _dot(el[...], w1ct[...]) + b1[...]
    m = jax.nn.relu(g1[...] + ce)
    m = jax.nn.relu(_dot(m, w2t[...]) + b2[...])
    msg_o[...] = _ln(m, gg[...], gbe[...])


def _edge_step2(el, g1, w1ct, b1, w2t, b2, gg, gbe):
    nblk = NEDGE // EB
    eb_spec = pl.BlockSpec((EB, LAT), lambda i: (i, 0))
    w_spec = lambda s: pl.BlockSpec(s, lambda i: (0,) * len(s))
    return pl.pallas_call(
        _edge2_body,
        grid=(nblk,),
        in_specs=[eb_spec, eb_spec,
                  w_spec((LAT, LAT)), w_spec((1, LAT)), w_spec((LAT, LAT)),
                  w_spec((1, LAT)), w_spec((1, LAT)), w_spec((1, LAT))],
        out_specs=eb_spec,
        out_shape=jax.ShapeDtypeStruct((NEDGE, LAT), jnp.float32),
    )(el, g1, w1ct, b1, w2t, b2, gg, gbe)


def _node_upd_body(aggr, nl, wat, wbt, b1, w2t, b2, gg, gbe, w1at, w1bt,
                   nl_o, a_o, b_o):
    h = jax.nn.relu(_dot(aggr[...], wat[...]) + _dot(nl[...], wbt[...]) + b1[...])
    h = jax.nn.relu(_dot(h, w2t[...]) + b2[...])
    nl2 = _ln(h, gg[...], gbe[...]) + nl[...]
    nl_o[...] = nl2
    a_o[...] = _dot(nl2, w1at[...])
    b_o[...] = _dot(nl2, w1bt[...])


def _node_update(aggr, nl, wat, wbt, b1, w2t, b2, gg, gbe, w1at, w1bt):
    out = [jax.ShapeDtypeStruct((NNODE, LAT), jnp.float32)] * 3
    return pl.pallas_call(
        _node_upd_body,
        out_shape=out,
    )(aggr, nl, wat, wbt, b1, w2t, b2, gg, gbe, w1at, w1bt)


def _decode_body(aggr, nl, wat, wbt, b1, w2t, b2, gg, gbe,
                 g1wt, g1bi, g1hr, g1hz, g1hn,
                 g2wt, g2bi, g2hr, g2hz, g2hn,
                 dW1t, db1, dW2t, db2, out_o):
    h = jax.nn.relu(_dot(aggr[...], wat[...]) + _dot(nl[...], wbt[...]) + b1[...])
    h = jax.nn.relu(_dot(h, w2t[...]) + b2[...])
    nl2 = _ln(h, gg[...], gbe[...]) + nl[...]

    gi = _dot(nl2, g1wt[...]) + g1bi[...]
    r = jax.nn.sigmoid(gi[:, :LAT] + g1hr[...])
    z = jax.nn.sigmoid(gi[:, LAT:2 * LAT] + g1hz[...])
    nn = jnp.tanh(gi[:, 2 * LAT:] + r * g1hn[...])
    h1 = (1.0 - z) * nn

    gi2 = _dot(h1, g2wt[...]) + g2bi[...]
    r2 = jax.nn.sigmoid(gi2[:, :LAT] + g2hr[...])
    z2 = jax.nn.sigmoid(gi2[:, LAT:2 * LAT] + g2hz[...])
    nn2 = jnp.tanh(gi2[:, 2 * LAT:] + r2 * g2hn[...])
    h2 = (1.0 - z2) * nn2

    d = jax.nn.relu(_dot(h2, dW1t[...]) + db1[...])
    out_o[...] = _dot(d, dW2t[...]) + db2[...]


def _decode(aggr, nl, wat, wbt, b1, w2t, b2, gg, gbe,
            g1wt, g1bi, g1hr, g1hz, g1hn, g2wt, g2bi, g2hr, g2hz, g2hn,
            dW1t, db1, dW2t, db2):
    return pl.pallas_call(
        _decode_body,
        out_shape=jax.ShapeDtypeStruct((NNODE, LAT), jnp.float32),
    )(aggr, nl, wat, wbt, b1, w2t, b2, gg, gbe,
      g1wt, g1bi, g1hr, g1hz, g1hn, g2wt, g2bi, g2hr, g2hz, g2hn,
      dW1t, db1, dW2t, db2)


def kernel(x, edge_index, edge_attr, ne_W1, ne_b1, ne_W2, ne_b2, ne_g, ne_be,
           ee_W1, ee_b1, ee_W2, ee_b2, ee_g, ee_be,
           gbe_W1, gbe_b1, gbe_W2, gbe_b2, gbe_g, gbe_be,
           gbn_W1, gbn_b1, gbn_W2, gbn_b2, gbn_g, gbn_be,
           g1_Wih, g1_Whh, g1_bih, g1_bhh,
           g2_Wih, g2_Whh, g2_bih, g2_bhh,
           dec_W1, dec_b1, dec_W2, dec_b2):
    row = edge_index[0]
    col = edge_index[1]
    r2 = lambda v: v.reshape(1, -1)

    w1at = gbe_W1[:, :LAT].T
    w1bt = gbe_W1[:, LAT:2 * LAT].T
    w1ct = gbe_W1[:, 2 * LAT:].T
    wat = gbn_W1[:, :LAT].T
    wbt = gbn_W1[:, LAT:].T

    nl, A, B = _encode_nodes(x, ne_W1.T, r2(ne_b1), ne_W2.T, r2(ne_b2),
                             r2(ne_g), r2(ne_be), w1at, w1bt)

    G1 = jnp.take(A, col, axis=0) + jnp.take(B, row, axis=0)
    G2 = jnp.take(A, row, axis=0) + jnp.take(B, col, axis=0)
    msg, el = _edge_step1(edge_attr, G1, G2, ee_W1.T, r2(ee_b1), ee_W2.T,
                          r2(ee_b2), r2(ee_g), r2(ee_be),
                          w1ct, r2(gbe_b1), gbe_W2.T, r2(gbe_b2),
                          r2(gbe_g), r2(gbe_be))
    aggr = jnp.zeros((NNODE, LAT), jnp.float32).at[col].add(msg)
    nl, A, B = _node_update(aggr, nl, wat, wbt, r2(gbn_b1), gbn_W2.T,
                            r2(gbn_b2), r2(gbn_g), r2(gbn_be), w1at, w1bt)

    G1 = jnp.take(A, col, axis=0) + jnp.take(B, row, axis=0)
    msg = _edge_step2(el, G1, w1ct, r2(gbe_b1), gbe_W2.T, r2(gbe_b2),
                      r2(gbe_g), r2(gbe_be))
    aggr = jnp.zeros((NNODE, LAT), jnp.float32).at[col].add(msg)

    g1hr, g1hz, g1hn = g1_bhh[:LAT], g1_bhh[LAT:2 * LAT], g1_bhh[2 * LAT:]
    g2hr, g2hz, g2hn = g2_bhh[:LAT], g2_bhh[LAT:2 * LAT], g2_bhh[2 * LAT:]
    dW2t = jnp.zeros((LAT, LAT), jnp.float32).at[:, :3].set(dec_W2.T)
    db2 = jnp.zeros((1, LAT), jnp.float32).at[:, :3].set(dec_b2)
    dec = _decode(aggr, nl, wat, wbt, r2(gbn_b1), gbn_W2.T, r2(gbn_b2),
                  r2(gbn_g), r2(gbn_be),
                  g1_Wih.T, r2(g1_bih), r2(g1hr), r2(g1hz), r2(g1hn),
                  g2_Wih.T, r2(g2_bih), r2(g2hr), r2(g2hz), r2(g2hn),
                  dec_W1.T, r2(dec_b1), dW2t, db2)
    return dec[None, :, :3]


# speedup vs baseline: 4.4126x; 4.4126x over previous
import functools

import jax
import jax.numpy as jnp
from jax import lax
from jax.experimental import pallas as pl
from jax.experimental.pallas import tpu as pltpu
from jax.experimental.pallas import tpu_sc as plsc

LAT = 128
NNODE = 10000
NEDGE = 320000
EB = 3200
NC = 2
NW = 32
EPW = NEDGE // NW
CH = 80
NCH = EPW // CH

_MESH = plsc.VectorSubcoreMesh(core_axis_name="c", subcore_axis_name="s")


def _ln(h, g, b):
    m = jnp.mean(h, axis=-1, keepdims=True)
    v = jnp.mean((h - m) ** 2, axis=-1, keepdims=True)
    return (h - m) * jax.lax.rsqrt(v + 1e-5) * g + b


def _dot(a, b):
    return jnp.dot(a, b, preferred_element_type=jnp.float32)


def _sc_gather_pair(nl, col3, row3):
    @functools.partial(
        pl.kernel,
        mesh=_MESH,
        out_type=[jax.ShapeDtypeStruct((NEDGE, LAT), jnp.float32)] * 2,
        scratch_types=[
            pltpu.VMEM((NCH, CH), jnp.int32),
            pltpu.VMEM((NCH, CH), jnp.int32),
            pltpu.VMEM((2, CH, LAT), jnp.float32),
            pltpu.VMEM((2, CH, LAT), jnp.float32),
            pltpu.SemaphoreType.DMA,
            pltpu.SemaphoreType.DMA,
            pltpu.SemaphoreType.DMA,
            pltpu.SemaphoreType.DMA,
        ],
    )
    def k(nl_h, col_h, row_h, outc_h, outr_h, idxc, idxr, bufc, bufr,
          sc0, sc1, sr0, sr1):
        wid = lax.axis_index("s") * NC + lax.axis_index("c")
        base = wid * EPW
        pltpu.sync_copy(col_h.at[wid], idxc)
        pltpu.sync_copy(row_h.at[wid], idxr)
        csems = (sc0, sc1)
        rsems = (sr0, sr1)

        def start(j, b):
            pltpu.async_copy(nl_h.at[idxc.at[j]], bufc.at[b], csems[b])
            pltpu.async_copy(nl_h.at[idxr.at[j]], bufr.at[b], rsems[b])

        def finish(j, b):
            pltpu.make_async_copy(nl_h.at[idxc.at[j]], bufc.at[b],
                                  csems[b]).wait()
            pltpu.make_async_copy(nl_h.at[idxr.at[j]], bufr.at[b],
                                  rsems[b]).wait()
            pltpu.sync_copy(bufc.at[b], outc_h.at[pl.ds(base + j * CH, CH)])
            pltpu.sync_copy(bufr.at[b], outr_h.at[pl.ds(base + j * CH, CH)])

        start(0, 0)
        start(1, 1)

        def body(i, carry):
            j0 = i * 2
            for b in range(2):
                j = j0 + b
                finish(j, b)

                @pl.when(j + 2 < NCH)
                def _():
                    start(j + 2, b)
            return carry

        lax.fori_loop(0, (NCH - 1) // 2, body, 0)
        finish(NCH - 1, 0)

    return k(nl, col3, row3)


def _sc_scatter(msg, col3, zrows):
    @functools.partial(
        pl.kernel,
        mesh=_MESH,
        out_type=jax.ShapeDtypeStruct((NC, NNODE, LAT), jnp.float32),
        scratch_types=[
            pltpu.VMEM((NCH, CH), jnp.int32),
            pltpu.VMEM((2, CH, LAT), jnp.float32),
            pltpu.VMEM_SHARED((NNODE, LAT), jnp.float32),
            pltpu.SemaphoreType.DMA,
            pltpu.SemaphoreType.DMA,
        ],
    )
    def k(msg_h, col_h, z_h, out_h, idxc, mbuf, acc, s0, s1):
        sid = lax.axis_index("s")
        cid = lax.axis_index("c")
        wid = sid * NC + cid
        base = wid * EPW
        pltpu.sync_copy(col_h.at[wid], idxc)

        def zbody(t, carry):
            j = t * 16 + sid

            @pl.when(j < NCH)
            def _():
                pltpu.sync_copy(z_h, acc.at[pl.ds(j * CH, CH)])
            return carry

        lax.fori_loop(0, 8, zbody, 0)
        plsc.subcore_barrier()

        sems = (s0, s1)

        def load(j, b):
            pltpu.async_copy(msg_h.at[pl.ds(base + j * CH, CH)], mbuf.at[b],
                             sems[b])

        def finish(j, b):
            pltpu.make_async_copy(msg_h.at[pl.ds(base + j * CH, CH)],
                                  mbuf.at[b], sems[b]).wait()
            pltpu.sync_copy(mbuf.at[b], acc.at[idxc.at[j]], add=True)

        load(0, 0)
        load(1, 1)

        def body(i, carry):
            j0 = i * 2
            for b in range(2):
                j = j0 + b
                finish(j, b)

                @pl.when(j + 2 < NCH)
                def _():
                    load(j + 2, b)
            return carry

        lax.fori_loop(0, (NCH - 1) // 2, body, 0)
        finish(NCH - 1, 0)
        plsc.subcore_barrier()

        def wbody(t, carry):
            j = t * 16 + sid

            @pl.when(j < NCH)
            def _():
                pltpu.sync_copy(acc.at[pl.ds(j * CH, CH)],
                                out_h.at[cid].at[pl.ds(j * CH, CH)])
            return carry

        lax.fori_loop(0, 8, wbody, 0)

    return k(msg, col3, zrows)


def _encode_nodes_body(x, neW1t, neb1, neW2t, neb2, neg, nebe, nl_o):
    h = jax.nn.relu(_dot(x[...], neW1t[...]) + neb1[...])
    h = jax.nn.relu(_dot(h, neW2t[...]) + neb2[...])
    nl_o[...] = _ln(h, neg[...], nebe[...])


def _encode_nodes(x, neW1t, neb1, neW2t, neb2, neg, nebe):
    return pl.pallas_call(
        _encode_nodes_body,
        out_shape=jax.ShapeDtypeStruct((NNODE, LAT), jnp.float32),
    )(x, neW1t, neb1, neW2t, neb2, neg, nebe)


def _edge1_body(ea, nlc, nlr, eeW1t, eeb1, eeW2t, eeb2, eeg, eebe,
                w1at, w1bt, w1ct, b1, w2t, b2, gg, gbe, msg_o, el_o):
    h = jax.nn.relu(_dot(ea[...], eeW1t[...]) + eeb1[...])
    h = jax.nn.relu(_dot(h, eeW2t[...]) + eeb2[...])
    el = _ln(h, eeg[...], eebe[...])
    ce = _dot(el, w1ct[...]) + b1[...]
    tc = _dot(nlc[...], w1at[...])
    tr = _dot(nlr[...], w1bt[...])
    m = jax.nn.relu(tc + tr + ce)
    m = jax.nn.relu(_dot(m, w2t[...]) + b2[...])
    msg_o[...] = _ln(m, gg[...], gbe[...])
    n = jax.nn.relu(_dot(nlr[...], w1at[...]) + _dot(nlc[...], w1bt[...]) + ce)
    n = jax.nn.relu(_dot(n, w2t[...]) + b2[...])
    el_o[...] = _ln(n, gg[...], gbe[...]) + el


def _edge_step1(ea, nlc, nlr, eeW1t, eeb1, eeW2t, eeb2, eeg, eebe,
                w1at, w1bt, w1ct, b1, w2t, b2, gg, gbe):
    nblk = NEDGE // EB
    eb_spec = pl.BlockSpec((EB, LAT), lambda i: (i, 0))
    ea_spec = pl.BlockSpec((EB, 16), lambda i: (i, 0))
    w_spec = lambda s: pl.BlockSpec(s, lambda i: (0,) * len(s))
    return pl.pallas_call(
        _edge1_body,
        grid=(nblk,),
        in_specs=[ea_spec, eb_spec, eb_spec,
                  w_spec((16, LAT)), w_spec((1, LAT)), w_spec((LAT, LAT)),
                  w_spec((1, LAT)), w_spec((1, LAT)), w_spec((1, LAT)),
                  w_spec((LAT, LAT)), w_spec((LAT, LAT)), w_spec((LAT, LAT)),
                  w_spec((1, LAT)), w_spec((LAT, LAT)), w_spec((1, LAT)),
                  w_spec((1, LAT)), w_spec((1, LAT))],
        out_specs=[eb_spec, eb_spec],
        out_shape=[jax.ShapeDtypeStruct((NEDGE, LAT), jnp.float32)] * 2,
    )(ea, nlc, nlr, eeW1t, eeb1, eeW2t, eeb2, eeg, eebe,
      w1at, w1bt, w1ct, b1, w2t, b2, gg, gbe)


def _edge2_body(el, nlc, nlr, w1at, w1bt, w1ct, b1, w2t, b2, gg, gbe, msg_o):
    ce = _dot(el[...], w1ct[...]) + b1[...]
    m = jax.nn.relu(_dot(nlc[...], w1at[...]) + _dot(nlr[...], w1bt[...]) + ce)
    m = jax.nn.relu(_dot(m, w2t[...]) + b2[...])
    msg_o[...] = _ln(m, gg[...], gbe[...])


def _edge_step2(el, nlc, nlr, w1at, w1bt, w1ct, b1, w2t, b2, gg, gbe):
    nblk = NEDGE // EB
    eb_spec = pl.BlockSpec((EB, LAT), lambda i: (i, 0))
    w_spec = lambda s: pl.BlockSpec(s, lambda i: (0,) * len(s))
    return pl.pallas_call(
        _edge2_body,
        grid=(nblk,),
        in_specs=[eb_spec, eb_spec, eb_spec,
                  w_spec((LAT, LAT)), w_spec((LAT, LAT)), w_spec((LAT, LAT)),
                  w_spec((1, LAT)), w_spec((LAT, LAT)), w_spec((1, LAT)),
                  w_spec((1, LAT)), w_spec((1, LAT))],
        out_specs=eb_spec,
        out_shape=jax.ShapeDtypeStruct((NEDGE, LAT), jnp.float32),
    )(el, nlc, nlr, w1at, w1bt, w1ct, b1, w2t, b2, gg, gbe)


def _node_upd_body(p2, nl, wat, wbt, b1, w2t, b2, gg, gbe, nl_o):
    aggr = p2[0] + p2[1]
    h = jax.nn.relu(_dot(aggr, wat[...]) + _dot(nl[...], wbt[...]) + b1[...])
    h = jax.nn.relu(_dot(h, w2t[...]) + b2[...])
    nl_o[...] = _ln(h, gg[...], gbe[...]) + nl[...]


def _node_update(p2, nl, wat, wbt, b1, w2t, b2, gg, gbe):
    return pl.pallas_call(
        _node_upd_body,
        out_shape=jax.ShapeDtypeStruct((NNODE, LAT), jnp.float32),
    )(p2, nl, wat, wbt, b1, w2t, b2, gg, gbe)


def _decode_body(p2, nl, wat, wbt, b1, w2t, b2, gg, gbe,
                 g1wt, g1bi, g1hr, g1hz, g1hn,
                 g2wt, g2bi, g2hr, g2hz, g2hn,
                 dW1t, db1, dW2t, db2, out_o):
    aggr = p2[0] + p2[1]
    h = jax.nn.relu(_dot(aggr, wat[...]) + _dot(nl[...], wbt[...]) + b1[...])
    h = jax.nn.relu(_dot(h, w2t[...]) + b2[...])
    nl2 = _ln(h, gg[...], gbe[...]) + nl[...]

    gi = _dot(nl2, g1wt[...]) + g1bi[...]
    r = jax.nn.sigmoid(gi[:, :LAT] + g1hr[...])
    z = jax.nn.sigmoid(gi[:, LAT:2 * LAT] + g1hz[...])
    nn = jnp.tanh(gi[:, 2 * LAT:] + r * g1hn[...])
    h1 = (1.0 - z) * nn

    gi2 = _dot(h1, g2wt[...]) + g2bi[...]
    r2 = jax.nn.sigmoid(gi2[:, :LAT] + g2hr[...])
    z2 = jax.nn.sigmoid(gi2[:, LAT:2 * LAT] + g2hz[...])
    nn2 = jnp.tanh(gi2[:, 2 * LAT:] + r2 * g2hn[...])
    h2 = (1.0 - z2) * nn2

    d = jax.nn.relu(_dot(h2, dW1t[...]) + db1[...])
    out_o[...] = _dot(d, dW2t[...]) + db2[...]


def _decode(p2, nl, wat, wbt, b1, w2t, b2, gg, gbe,
            g1wt, g1bi, g1hr, g1hz, g1hn, g2wt, g2bi, g2hr, g2hz, g2hn,
            dW1t, db1, dW2t, db2):
    return pl.pallas_call(
        _decode_body,
        out_shape=jax.ShapeDtypeStruct((NNODE, LAT), jnp.float32),
    )(p2, nl, wat, wbt, b1, w2t, b2, gg, gbe,
      g1wt, g1bi, g1hr, g1hz, g1hn, g2wt, g2bi, g2hr, g2hz, g2hn,
      dW1t, db1, dW2t, db2)


def kernel(x, edge_index, edge_attr, ne_W1, ne_b1, ne_W2, ne_b2, ne_g, ne_be,
           ee_W1, ee_b1, ee_W2, ee_b2, ee_g, ee_be,
           gbe_W1, gbe_b1, gbe_W2, gbe_b2, gbe_g, gbe_be,
           gbn_W1, gbn_b1, gbn_W2, gbn_b2, gbn_g, gbn_be,
           g1_Wih, g1_Whh, g1_bih, g1_bhh,
           g2_Wih, g2_Whh, g2_bih, g2_bhh,
           dec_W1, dec_b1, dec_W2, dec_b2):
    row3 = edge_index[0].reshape(NW, NCH, CH)
    col3 = edge_index[1].reshape(NW, NCH, CH)
    r2 = lambda v: v.reshape(1, -1)
    zrows = jnp.zeros((CH, LAT), jnp.float32)

    w1at = gbe_W1[:, :LAT].T
    w1bt = gbe_W1[:, LAT:2 * LAT].T
    w1ct = gbe_W1[:, 2 * LAT:].T
    wat = gbn_W1[:, :LAT].T
    wbt = gbn_W1[:, LAT:].T

    nl = _encode_nodes(x, ne_W1.T, r2(ne_b1), ne_W2.T, r2(ne_b2),
                       r2(ne_g), r2(ne_be))

    nlc, nlr = _sc_gather_pair(nl, col3, row3)
    msg, el = _edge_step1(edge_attr, nlc, nlr, ee_W1.T, r2(ee_b1), ee_W2.T,
                          r2(ee_b2), r2(ee_g), r2(ee_be),
                          w1at, w1bt, w1ct, r2(gbe_b1), gbe_W2.T, r2(gbe_b2),
                          r2(gbe_g), r2(gbe_be))
    p2 = _sc_scatter(msg, col3, zrows)
    nl = _node_update(p2, nl, wat, wbt, r2(gbn_b1), gbn_W2.T,
                      r2(gbn_b2), r2(gbn_g), r2(gbn_be))

    nlc, nlr = _sc_gather_pair(nl, col3, row3)
    msg = _edge_step2(el, nlc, nlr, w1at, w1bt, w1ct, r2(gbe_b1), gbe_W2.T,
                      r2(gbe_b2), r2(gbe_g), r2(gbe_be))
    p2 = _sc_scatter(msg, col3, zrows)

    g1hr, g1hz, g1hn = g1_bhh[:LAT], g1_bhh[LAT:2 * LAT], g1_bhh[2 * LAT:]
    g2hr, g2hz, g2hn = g2_bhh[:LAT], g2_bhh[LAT:2 * LAT], g2_bhh[2 * LAT:]
    dW2t = jnp.zeros((LAT, LAT), jnp.float32).at[:, :3].set(dec_W2.T)
    db2 = jnp.zeros((1, LAT), jnp.float32).at[:, :3].set(dec_b2)
    dec = _decode(p2, nl, wat, wbt, r2(gbn_b1), gbn_W2.T, r2(gbn_b2),
                  r2(gbn_g), r2(gbn_be),
                  g1_Wih.T, r2(g1_bih), r2(g1hr), r2(g1hz), r2(g1hn),
                  g2_Wih.T, r2(g2_bih), r2(g2hr), r2(g2hz), r2(g2hn),
                  dec_W1.T, r2(dec_b1), dW2t, db2)
    return dec[None, :, :3]
